# baseline (speedup 1.0000x reference)
import math

import jax
import jax.numpy as jnp
from jax.experimental import pallas as pl
from jax.experimental.pallas import tpu as pltpu

_NOISE_STD = 1.0
_GSHARD_W = 0.0
_IMP_W = 1.0
_LOAD_W = 1.0


def _router_kernel(x_ref, w_ref, gamma_ref, beta_ref, noise_ref,
                   gates_out_ref, stats_ref,
                   wa_s, cb_s, imp_acc, mg_acc, cnt_acc, lsum_acc, lsq_acc):
    i = pl.program_id(0)
    nsteps = pl.num_programs(0)
    g, bs, d = x_ref.shape
    e = w_ref.shape[0]
    rows = g * bs
    noise_std = max(1.0 / e * _NOISE_STD, 1e-6)

    @pl.when(i == 0)
    def _():
        wg0 = w_ref[...] * gamma_ref[...]
        wa_s[:e, :] = wg0
        wa_s[e:, :] = jnp.ones((1, d), jnp.float32)
        cb_s[0:1, :] = jnp.sum(wg0, axis=1).reshape(1, e)
        cb_s[1:2, :] = jnp.sum(w_ref[...] * beta_ref[...],
                               axis=1).reshape(1, e)

    x = x_ref[...]
    xm = x.reshape(rows, d)
    ya = jax.lax.dot_general(
        xm, wa_s[...], (((1,), (1,)), ((), ())),
        preferred_element_type=jnp.float32)
    s2 = jax.lax.dot_general(
        xm * xm, jnp.ones((1, d), jnp.float32), (((1,), (1,)), ((), ())),
        preferred_element_type=jnp.float32)
    mu = ya[:, e:e + 1] * (1.0 / d)
    var = s2 * (1.0 / d) - mu * mu
    inv = jax.lax.rsqrt(var + 1e-5)
    logits = (ya[:, :e] - mu * cb_s[0:1, :]) * inv + cb_s[1:2, :]

    eg = jnp.exp(logits)
    gates = eg / jnp.sum(eg, axis=-1, keepdims=True)

    ln = logits + noise_std * noise_ref[...].reshape(rows, e)
    en = jnp.exp(ln)
    gates_noisy = en / jnp.sum(en, axis=-1, keepdims=True)
    gates_out_ref[...] = gates_noisy.reshape(g, bs, e)

    iota = jax.lax.broadcasted_iota(jnp.int32, (rows, e), 1)
    m1 = jnp.max(ln, axis=-1, keepdims=True)
    a1 = jnp.min(jnp.where(ln >= m1, iota, e), axis=-1, keepdims=True)
    thr = jnp.max(jnp.where(iota == a1, -jnp.inf, ln), axis=-1, keepdims=True)
    nrw = jnp.clip((thr - logits) / noise_std, -10.0, 10.0)
    p = 0.5 * (1.0 + jax.lax.erf(nrw * (1.0 / math.sqrt(2.0))))
    pm = jnp.mean(p.reshape(g, bs, e), axis=0)

    imp_part = jnp.sum(gates.reshape(g, bs, e), axis=1)
    mg_part = jnp.sum(gates_noisy, axis=0, keepdims=True)
    cnt_part = jnp.sum((iota == a1).astype(jnp.float32), axis=0,
                       keepdims=True)
    lsum_part = jnp.sum(pm)
    lsq_part = jnp.sum(pm * pm)

    @pl.when(i == 0)
    def _():
        imp_acc[...] = imp_part
        mg_acc[...] = mg_part
        cnt_acc[...] = cnt_part
        lsum_acc[0, 0] = lsum_part
        lsq_acc[0, 0] = lsq_part

    @pl.when(i > 0)
    def _():
        imp_acc[...] += imp_part
        mg_acc[...] += mg_part
        cnt_acc[...] += cnt_part
        lsum_acc[0, 0] += lsum_part
        lsq_acc[0, 0] += lsq_part

    @pl.when(i == nsteps - 1)
    def _():
        n_tok = jnp.float32(g * bs * nsteps)
        imp = imp_acc[...]
        imp_mean = jnp.mean(imp, axis=1, keepdims=True)
        imp_var = jnp.sum((imp - imp_mean) ** 2, axis=1,
                          keepdims=True) / (e - 1)
        imp_loss = jnp.mean(imp_var / (imp_mean * imp_mean))

        mean_t = cnt_acc[...] / n_tok
        mean_g = mg_acc[...] / n_tok
        gshard = jnp.mean(mean_t * mean_g) * (e * e)

        m = jnp.float32(bs * nsteps * e)
        pm_mean = lsum_acc[0, 0] / m
        pm_var = lsq_acc[0, 0] / m - pm_mean * pm_mean
        load = pm_var / (pm_mean * pm_mean)

        stats_ref[0, 0] = _GSHARD_W * gshard + _IMP_W * imp_loss + _LOAD_W * load
        stats_ref[0, 1] = gshard
        stats_ref[0, 2] = imp_loss
        stats_ref[0, 3] = load


def kernel(inputs, W, gamma, beta, noise):
    g, s, d = inputs.shape
    e = W.shape[0]
    bs = 512
    grid = (s // bs,)

    gates_noisy, stats = pl.pallas_call(
        _router_kernel,
        grid=grid,
        in_specs=[
            pl.BlockSpec((g, bs, d), lambda i: (0, i, 0)),
            pl.BlockSpec((e, d), lambda i: (0, 0)),
            pl.BlockSpec((1, d), lambda i: (0, 0)),
            pl.BlockSpec((1, d), lambda i: (0, 0)),
            pl.BlockSpec((g, bs, e), lambda i: (0, i, 0)),
        ],
        out_specs=[
            pl.BlockSpec((g, bs, e), lambda i: (0, i, 0)),
            pl.BlockSpec(memory_space=pltpu.SMEM),
        ],
        out_shape=[
            jax.ShapeDtypeStruct((g, s, e), jnp.float32),
            jax.ShapeDtypeStruct((1, 4), jnp.float32),
        ],
        scratch_shapes=[
            pltpu.VMEM((e + 1, d), jnp.float32),
            pltpu.VMEM((2, e), jnp.float32),
            pltpu.VMEM((g, e), jnp.float32),
            pltpu.VMEM((1, e), jnp.float32),
            pltpu.VMEM((1, e), jnp.float32),
            pltpu.SMEM((1, 1), jnp.float32),
            pltpu.SMEM((1, 1), jnp.float32),
        ],
    )(inputs, W, gamma.reshape(1, d), beta.reshape(1, d), noise)

    return (gates_noisy, stats[0, 0], stats[0, 1], stats[0, 2], stats[0, 3])


# speedup vs baseline: 1.9537x; 1.9537x over previous
import math

import jax
import jax.numpy as jnp
from jax.experimental import pallas as pl
from jax.experimental.pallas import tpu as pltpu

_NOISE_STD = 1.0
_GSHARD_W = 0.0
_IMP_W = 1.0
_LOAD_W = 1.0


def _router_kernel(x_ref, w_ref, gamma_ref, beta_ref, noise_ref,
                   gates_out_ref, stats_ref,
                   wa_s, cb_s, imp_acc, mg_acc, cnt_acc, lsum_acc, lsq_acc):
    i = pl.program_id(0)
    nsteps = pl.num_programs(0)
    g, bs, d = x_ref.shape
    e = w_ref.shape[0]
    noise_std = max(1.0 / e * _NOISE_STD, 1e-6)

    @pl.when(i == 0)
    def _():
        wg0 = w_ref[...] * gamma_ref[...]
        wa_s[:e, :] = wg0
        wa_s[e:, :] = jnp.ones((1, d), jnp.float32)
        cb_s[:, 0:1] = jnp.sum(wg0, axis=1, keepdims=True)
        cb_s[:, 1:2] = jnp.sum(w_ref[...] * beta_ref[...],
                               axis=1, keepdims=True)

    iota = jax.lax.broadcasted_iota(jnp.int32, (e, bs), 0)
    cs = cb_s[:, 0:1]
    bw = cb_s[:, 1:2]
    wa = wa_s[...]
    ones_row = jnp.ones((1, d), jnp.float32)

    p_sum = jnp.zeros((e, bs), jnp.float32)
    imp_part = []
    mg_part = jnp.zeros((e, 1), jnp.float32)
    cnt_part = jnp.zeros((e, 1), jnp.float32)

    for gi in range(g):
        x_g = x_ref[gi]
        ya = jax.lax.dot_general(
            wa, x_g, (((1,), (1,)), ((), ())),
            preferred_element_type=jnp.float32)
        s2 = jax.lax.dot_general(
            ones_row, x_g * x_g, (((1,), (1,)), ((), ())),
            preferred_element_type=jnp.float32)
        mu = ya[e:e + 1, :] * (1.0 / d)
        var = s2 * (1.0 / d) - mu * mu
        inv = jax.lax.rsqrt(var + 1e-5)
        logits = (ya[:e, :] - cs * mu) * inv + bw

        eg = jnp.exp(logits)
        gates = eg / jnp.sum(eg, axis=0, keepdims=True)

        ln = logits + noise_std * noise_ref[gi]
        en = jnp.exp(ln)
        gates_noisy = en / jnp.sum(en, axis=0, keepdims=True)
        gates_out_ref[gi] = gates_noisy

        m1 = jnp.max(ln, axis=0, keepdims=True)
        a1 = jnp.min(jnp.where(ln >= m1, iota, e), axis=0, keepdims=True)
        oh = (iota == a1)
        thr = jnp.max(jnp.where(oh, -jnp.inf, ln), axis=0, keepdims=True)
        nrw = jnp.clip((thr - logits) * (1.0 / noise_std), -10.0, 10.0)
        p_sum = p_sum + 0.5 * (1.0 + jax.lax.erf(nrw * (1.0 / math.sqrt(2.0))))

        imp_part.append(jnp.sum(gates, axis=1, keepdims=True))
        mg_part = mg_part + jnp.sum(gates_noisy, axis=1, keepdims=True)
        cnt_part = cnt_part + jnp.sum(oh.astype(jnp.float32), axis=1,
                                      keepdims=True)

    pm = p_sum * (1.0 / g)
    lsum_part = jnp.sum(pm)
    lsq_part = jnp.sum(pm * pm)
    imp_part = jnp.concatenate(imp_part, axis=1)

    @pl.when(i == 0)
    def _():
        imp_acc[...] = imp_part
        mg_acc[...] = mg_part
        cnt_acc[...] = cnt_part
        lsum_acc[0, 0] = lsum_part
        lsq_acc[0, 0] = lsq_part

    @pl.when(i > 0)
    def _():
        imp_acc[...] += imp_part
        mg_acc[...] += mg_part
        cnt_acc[...] += cnt_part
        lsum_acc[0, 0] += lsum_part
        lsq_acc[0, 0] += lsq_part

    @pl.when(i == nsteps - 1)
    def _():
        n_tok = jnp.float32(g * bs * nsteps)
        imp = imp_acc[...]
        imp_mean = jnp.mean(imp, axis=0, keepdims=True)
        imp_var = jnp.sum((imp - imp_mean) ** 2, axis=0,
                          keepdims=True) / (e - 1)
        imp_loss = jnp.mean(imp_var / (imp_mean * imp_mean))

        mean_t = cnt_acc[...] / n_tok
        mean_g = mg_acc[...] / n_tok
        gshard = jnp.mean(mean_t * mean_g) * (e * e)

        m = jnp.float32(bs * nsteps * e)
        pm_mean = lsum_acc[0, 0] / m
        pm_var = lsq_acc[0, 0] / m - pm_mean * pm_mean
        load = pm_var / (pm_mean * pm_mean)

        stats_ref[0, 0] = _GSHARD_W * gshard + _IMP_W * imp_loss + _LOAD_W * load
        stats_ref[0, 1] = gshard
        stats_ref[0, 2] = imp_loss
        stats_ref[0, 3] = load


def kernel(inputs, W, gamma, beta, noise):
    g, s, d = inputs.shape
    e = W.shape[0]
    bs = 512
    grid = (s // bs,)

    noise_t = jnp.swapaxes(noise, 1, 2)
    gates_t, stats = pl.pallas_call(
        _router_kernel,
        grid=grid,
        in_specs=[
            pl.BlockSpec((g, bs, d), lambda i: (0, i, 0)),
            pl.BlockSpec((e, d), lambda i: (0, 0)),
            pl.BlockSpec((1, d), lambda i: (0, 0)),
            pl.BlockSpec((1, d), lambda i: (0, 0)),
            pl.BlockSpec((g, e, bs), lambda i: (0, 0, i)),
        ],
        out_specs=[
            pl.BlockSpec((g, e, bs), lambda i: (0, 0, i)),
            pl.BlockSpec(memory_space=pltpu.SMEM),
        ],
        out_shape=[
            jax.ShapeDtypeStruct((g, e, s), jnp.float32),
            jax.ShapeDtypeStruct((1, 4), jnp.float32),
        ],
        scratch_shapes=[
            pltpu.VMEM((e + 1, d), jnp.float32),
            pltpu.VMEM((e, 2), jnp.float32),
            pltpu.VMEM((e, g), jnp.float32),
            pltpu.VMEM((e, 1), jnp.float32),
            pltpu.VMEM((e, 1), jnp.float32),
            pltpu.SMEM((1, 1), jnp.float32),
            pltpu.SMEM((1, 1), jnp.float32),
        ],
    )(inputs, W, gamma.reshape(1, d), beta.reshape(1, d), noise_t)

    gates_noisy = jnp.swapaxes(gates_t, 1, 2)
    return (gates_noisy, stats[0, 0], stats[0, 1], stats[0, 2], stats[0, 3])


# speedup vs baseline: 2.1644x; 1.1079x over previous
import math

import jax
import jax.numpy as jnp
from jax.experimental import pallas as pl
from jax.experimental.pallas import tpu as pltpu

_NOISE_STD = 1.0
_GSHARD_W = 0.0
_IMP_W = 1.0
_LOAD_W = 1.0


def _router_kernel(x_ref, w_ref, gamma_ref, beta_ref, noise_ref,
                   gates_out_ref, stats_ref,
                   wa_s, cb_s, imp_acc, mg_acc, cnt_acc, lsum_acc, lsq_acc):
    i = pl.program_id(0)
    nsteps = pl.num_programs(0)
    g, bs, d = x_ref.shape
    e = w_ref.shape[0]
    noise_std = max(1.0 / e * _NOISE_STD, 1e-6)

    @pl.when(i == 0)
    def _():
        wg0 = w_ref[...] * gamma_ref[...]
        wa_s[:e, :] = wg0
        wa_s[e:, :] = jnp.ones((1, d), jnp.float32)
        cb_s[:, 0:1] = jnp.sum(wg0, axis=1, keepdims=True)
        cb_s[:, 1:2] = jnp.sum(w_ref[...] * beta_ref[...],
                               axis=1, keepdims=True)

    iota = jax.lax.broadcasted_iota(jnp.int32, (e, bs), 0)
    cs = cb_s[:, 0:1]
    bw = cb_s[:, 1:2]
    wa = wa_s[...]
    ones_row = jnp.ones((1, d), jnp.float32)

    p_sum = jnp.zeros((e, bs), jnp.float32)
    imp_part = []
    mg_part = jnp.zeros((e, 1), jnp.float32)
    cnt_part = jnp.zeros((e, 1), jnp.float32)

    for gi in range(g):
        x_g = x_ref[gi]
        ya = jax.lax.dot_general(
            wa, x_g, (((1,), (1,)), ((), ())),
            preferred_element_type=jnp.float32)
        s2 = jax.lax.dot_general(
            ones_row, x_g * x_g, (((1,), (1,)), ((), ())),
            preferred_element_type=jnp.float32)
        mu = ya[e:e + 1, :] * (1.0 / d)
        var = s2 * (1.0 / d) - mu * mu
        inv = jax.lax.rsqrt(var + 1e-5)
        logits = (ya[:e, :] - cs * mu) * inv + bw

        eg = jnp.exp(logits)
        gates = eg / jnp.sum(eg, axis=0, keepdims=True)

        ln = logits + noise_std * noise_ref[gi]
        en = jnp.exp(ln)
        gates_noisy = en / jnp.sum(en, axis=0, keepdims=True)
        gates_out_ref[gi] = gates_noisy

        m1 = jnp.max(ln, axis=0, keepdims=True)
        a1 = jnp.min(jnp.where(ln >= m1, iota, e), axis=0, keepdims=True)
        oh = (iota == a1)
        thr = jnp.max(jnp.where(oh, -jnp.inf, ln), axis=0, keepdims=True)
        nrw = jnp.clip((thr - logits) * (1.0 / noise_std), -10.0, 10.0)
        p_sum = p_sum + 0.5 * (1.0 + jax.lax.erf(nrw * (1.0 / math.sqrt(2.0))))

        imp_part.append(jnp.sum(gates, axis=1, keepdims=True))
        mg_part = mg_part + jnp.sum(gates_noisy, axis=1, keepdims=True)
        cnt_part = cnt_part + jnp.sum(oh.astype(jnp.float32), axis=1,
                                      keepdims=True)

    pm = p_sum * (1.0 / g)
    lsum_part = jnp.sum(pm)
    lsq_part = jnp.sum(pm * pm)
    imp_part = jnp.concatenate(imp_part, axis=1)

    @pl.when(i == 0)
    def _():
        imp_acc[...] = imp_part
        mg_acc[...] = mg_part
        cnt_acc[...] = cnt_part
        lsum_acc[0, 0] = lsum_part
        lsq_acc[0, 0] = lsq_part

    @pl.when(i > 0)
    def _():
        imp_acc[...] += imp_part
        mg_acc[...] += mg_part
        cnt_acc[...] += cnt_part
        lsum_acc[0, 0] += lsum_part
        lsq_acc[0, 0] += lsq_part

    @pl.when(i == nsteps - 1)
    def _():
        n_tok = jnp.float32(g * bs * nsteps)
        imp = imp_acc[...]
        imp_mean = jnp.mean(imp, axis=0, keepdims=True)
        imp_var = jnp.sum((imp - imp_mean) ** 2, axis=0,
                          keepdims=True) / (e - 1)
        imp_loss = jnp.mean(imp_var / (imp_mean * imp_mean))

        mean_t = cnt_acc[...] / n_tok
        mean_g = mg_acc[...] / n_tok
        gshard = jnp.mean(mean_t * mean_g) * (e * e)

        m = jnp.float32(bs * nsteps * e)
        pm_mean = lsum_acc[0, 0] / m
        pm_var = lsq_acc[0, 0] / m - pm_mean * pm_mean
        load = pm_var / (pm_mean * pm_mean)

        stats_ref[0, 0] = _GSHARD_W * gshard + _IMP_W * imp_loss + _LOAD_W * load
        stats_ref[0, 1] = gshard
        stats_ref[0, 2] = imp_loss
        stats_ref[0, 3] = load


def kernel(inputs, W, gamma, beta, noise):
    g, s, d = inputs.shape
    e = W.shape[0]
    bs = 1024
    grid = (s // bs,)

    noise_t = jnp.swapaxes(noise, 1, 2)
    gates_t, stats = pl.pallas_call(
        _router_kernel,
        grid=grid,
        in_specs=[
            pl.BlockSpec((g, bs, d), lambda i: (0, i, 0)),
            pl.BlockSpec((e, d), lambda i: (0, 0)),
            pl.BlockSpec((1, d), lambda i: (0, 0)),
            pl.BlockSpec((1, d), lambda i: (0, 0)),
            pl.BlockSpec((g, e, bs), lambda i: (0, 0, i)),
        ],
        out_specs=[
            pl.BlockSpec((g, e, bs), lambda i: (0, 0, i)),
            pl.BlockSpec(memory_space=pltpu.SMEM),
        ],
        out_shape=[
            jax.ShapeDtypeStruct((g, e, s), jnp.float32),
            jax.ShapeDtypeStruct((1, 4), jnp.float32),
        ],
        scratch_shapes=[
            pltpu.VMEM((e + 1, d), jnp.float32),
            pltpu.VMEM((e, 2), jnp.float32),
            pltpu.VMEM((e, g), jnp.float32),
            pltpu.VMEM((e, 1), jnp.float32),
            pltpu.VMEM((e, 1), jnp.float32),
            pltpu.SMEM((1, 1), jnp.float32),
            pltpu.SMEM((1, 1), jnp.float32),
        ],
    )(inputs, W, gamma.reshape(1, d), beta.reshape(1, d), noise_t)

    gates_noisy = jnp.swapaxes(gates_t, 1, 2)
    return (gates_noisy, stats[0, 0], stats[0, 1], stats[0, 2], stats[0, 3])
